# Initial kernel scaffold; baseline (speedup 1.0000x reference)
import jax
import jax.numpy as jnp
from jax.experimental import pallas as pl


def kernel(x, table):
    raise NotImplementedError("write your pallas kernel here")


# speedup vs baseline: 1.6004x; 1.6004x over previous
import functools

import jax
import jax.numpy as jnp
import numpy as np
from jax import lax
from jax.experimental import pallas as pl
from jax.experimental.pallas import tpu as pltpu
from jax.experimental.pallas import tpu_sc as plsc

_VOCAB = 100000
_MAX_LEN = 2048
_D = 768
_B = 4

_NC = 2
_NS = 16
_NW = _NC * _NS
_P = _MAX_LEN // _NW
_LANES = 16
_CPR = _D // _LANES


def _pos_encoding_np(max_len: int, d_model: int) -> np.ndarray:
    pos = np.arange(max_len, dtype=np.float32)[:, None]
    two_i = np.arange(0, d_model, 2, dtype=np.float32)
    ang = pos / (np.float32(10000.0) ** (two_i / np.float32(d_model)))
    enc = np.zeros((max_len, d_model), dtype=np.float32)
    enc[:, 0::2] = np.sin(ang)
    enc[:, 1::2] = np.cos(ang)
    return enc


_ENC = _pos_encoding_np(_MAX_LEN, _D)


def _sc_body(x_hbm, table_hbm, enc_hbm, out_hbm, idx_v, enc_v, rows_v, sem):
    c = lax.axis_index("c")
    s = lax.axis_index("s")
    w = s * _NC + c
    pltpu.sync_copy(enc_hbm.at[pl.ds(w * _P, _P)], enc_v)
    for b in range(_B):
        pltpu.sync_copy(x_hbm.at[b, pl.ds(w * _P, _P)], idx_v.at[b])
    for b in range(_B):
        pltpu.async_copy(table_hbm.at[idx_v.at[b]], rows_v, sem).wait()

        @pl.loop(0, _P)
        def _row_add(r):
            for cc in range(_CPR):
                sl = pl.ds(cc * _LANES, _LANES)
                plsc.addupdate(rows_v.at[r, sl], enc_v[r, sl])

        pltpu.sync_copy(rows_v, out_hbm.at[pl.ds(b * _MAX_LEN + w * _P, _P)])


@functools.partial(jax.jit, static_argnames=())
def kernel(x, table):
    x32 = x.astype(jnp.int32)
    enc = jnp.asarray(_ENC)
    mesh = plsc.VectorSubcoreMesh(core_axis_name="c", subcore_axis_name="s")
    out = pl.kernel(
        _sc_body,
        out_type=jax.ShapeDtypeStruct((_B * _MAX_LEN, _D), jnp.float32),
        mesh=mesh,
        scratch_types=[
            pltpu.VMEM((_B, _P), jnp.int32),
            pltpu.VMEM((_P, _D), jnp.float32),
            pltpu.VMEM((_P, _D), jnp.float32),
            pltpu.SemaphoreType.DMA,
        ],
    )(x32, table, enc)
    return out.reshape(_B, _MAX_LEN, _D)


# speedup vs baseline: 1.6820x; 1.0510x over previous
import functools

import jax
import jax.numpy as jnp
import numpy as np
from jax import lax
from jax.experimental import pallas as pl
from jax.experimental.pallas import tpu as pltpu
from jax.experimental.pallas import tpu_sc as plsc

_VOCAB = 100000
_MAX_LEN = 2048
_D = 768
_B = 4

_NC = 2
_NS = 16
_NW = _NC * _NS
_P = _MAX_LEN // _NW
_LANES = 16
_CPR = _D // _LANES


def _pos_encoding_np(max_len: int, d_model: int) -> np.ndarray:
    pos = np.arange(max_len, dtype=np.float32)[:, None]
    two_i = np.arange(0, d_model, 2, dtype=np.float32)
    ang = pos / (np.float32(10000.0) ** (two_i / np.float32(d_model)))
    enc = np.zeros((max_len, d_model), dtype=np.float32)
    enc[:, 0::2] = np.sin(ang)
    enc[:, 1::2] = np.cos(ang)
    return enc


_ENC = _pos_encoding_np(_MAX_LEN, _D)


_S = 32
_NCHUNK = _B * _P // _S
_NBUF = 3


def _sc_body(x_hbm, table_hbm, enc_hbm, out_hbm, idx_v, enc_v,
             buf0, buf1, buf2, idx_sem, enc_sem,
             g0, g1, g2, s0, s1, s2):
    c = lax.axis_index("c")
    s = lax.axis_index("s")
    w = s * _NC + c
    bufs = (buf0, buf1, buf2)
    gsem = (g0, g1, g2)
    ssem = (s0, s1, s2)

    idescs = [pltpu.async_copy(x_hbm.at[b, pl.ds(w * _P, _P)],
                               idx_v.at[b], idx_sem) for b in range(_B)]
    for d in idescs:
        d.wait()
    edesc = pltpu.async_copy(enc_hbm.at[pl.ds(w * _P, _P)], enc_v, enc_sem)

    gdesc = [None] * _NCHUNK
    sdesc = [None] * _NCHUNK

    def fire_gather(i):
        j = i % _NBUF
        if i >= _NBUF:
            sdesc[i - _NBUF].wait()
        b, h = i // 2, i % 2
        gdesc[i] = pltpu.async_copy(
            table_hbm.at[idx_v.at[b, pl.ds(h * _S, _S)]], bufs[j], gsem[j])

    fire_gather(0)
    fire_gather(1)
    edesc.wait()
    for i in range(_NCHUNK):
        j = i % _NBUF
        b, h = i // 2, i % 2
        gdesc[i].wait()
        buf = bufs[j]

        @pl.loop(0, _S)
        def _row_add(r):
            for cc in range(_CPR):
                sl = pl.ds(cc * _LANES, _LANES)
                plsc.addupdate(buf.at[r, sl], enc_v[h * _S + r, sl])

        sdesc[i] = pltpu.async_copy(
            buf, out_hbm.at[pl.ds(b * _MAX_LEN + w * _P + h * _S, _S)],
            ssem[j])
        if i + 2 < _NCHUNK:
            fire_gather(i + 2)
    for i in range(_NCHUNK - _NBUF, _NCHUNK):
        sdesc[i].wait()


@functools.partial(jax.jit, static_argnames=())
def kernel(x, table):
    x32 = x.astype(jnp.int32)
    enc = jnp.asarray(_ENC)
    mesh = plsc.VectorSubcoreMesh(core_axis_name="c", subcore_axis_name="s")
    out = pl.kernel(
        _sc_body,
        out_type=jax.ShapeDtypeStruct((_B * _MAX_LEN, _D), jnp.float32),
        mesh=mesh,
        scratch_types=[
            pltpu.VMEM((_B, _P), jnp.int32),
            pltpu.VMEM((_P, _D), jnp.float32),
            pltpu.VMEM((_S, _D), jnp.float32),
            pltpu.VMEM((_S, _D), jnp.float32),
            pltpu.VMEM((_S, _D), jnp.float32),
            pltpu.SemaphoreType.DMA,
            pltpu.SemaphoreType.DMA,
            pltpu.SemaphoreType.DMA,
            pltpu.SemaphoreType.DMA,
            pltpu.SemaphoreType.DMA,
            pltpu.SemaphoreType.DMA,
            pltpu.SemaphoreType.DMA,
            pltpu.SemaphoreType.DMA,
        ],
    )(x32, table, enc)
    return out.reshape(_B, _MAX_LEN, _D)
